# baseline (speedup 1.0000x reference)
import functools

import jax
import jax.numpy as jnp
from jax import lax
from jax.experimental import pallas as pl
from jax.experimental.pallas import tpu as pltpu
from jax.experimental.pallas import tpu_sc as plsc

_INFO = plsc.get_sparse_core_info()
_NC = _INFO.num_cores
_NS = _INFO.num_subcores
_L = _INFO.num_lanes
_NW = _NC * _NS

_ROWS, _COLS = 16384, 200
_D = 4
_JH = _COLS // 8
_IH = _ROWS // 128
_NT = 8
_NUNITS = _JH * (_IH // _NT)
_MAXU = -(-_NUNITS // _NW)


def _make_emb():
    mesh = plsc.VectorSubcoreMesh(core_axis_name="c", subcore_axis_name="s")

    @functools.partial(
        pl.kernel,
        mesh=mesh,
        out_type=jax.ShapeDtypeStruct((_COLS, _IH, _D, 128), jnp.float32),
        compiler_params=pltpu.CompilerParams(needs_layout_passes=False),
        scratch_types=[
            pltpu.VMEM((8, _D), jnp.float32),
            pltpu.VMEM((2, _NT, 8, 128), jnp.int32),
            pltpu.VMEM((2, 8, _NT, _D, 128), jnp.float32),
            pltpu.SemaphoreType.DMA((2,)),
            pltpu.SemaphoreType.DMA((2,)),
        ],
    )
    def emb(tab_hbm, ids_hbm, out_hbm, tab_v, ids_v, out_v, isem, osem):
        wid = lax.axis_index("s") * _NC + lax.axis_index("c")
        pltpu.sync_copy(tab_hbm, tab_v)
        dvecs = [jnp.full((_L,), d, jnp.int32) for d in range(_D)]

        def valid(c):
            return c * _NW + wid < _NUNITS

        def ids_dma(c, buf):
            u = c * _NW + wid
            jh = u >> 4
            q = u & 15
            return pltpu.make_async_copy(
                ids_hbm.at[jh, pl.ds(q * _NT, _NT)], ids_v.at[buf], isem.at[buf]
            )

        def out_dma(c, buf):
            u = c * _NW + wid
            jh = u >> 4
            q = u & 15
            return pltpu.make_async_copy(
                out_v.at[buf],
                out_hbm.at[pl.ds(jh * 8, 8), pl.ds(q * _NT, _NT)],
                osem.at[buf],
            )

        ids_dma(0, 0).start()

        def unit_body(c, carry):
            cur = c & 1

            @pl.when(valid(c + 1))
            def _():
                ids_dma(c + 1, 1 - cur).start()

            @pl.when(valid(c))
            def _():
                ids_dma(c, cur).wait()

                @pl.when(c >= 2)
                def _():
                    out_dma(c - 2, cur).wait()

                @plsc.parallel_loop(0, _NT * 64, unroll=8)
                def body(t):
                    ti = t >> 6
                    jl = (t >> 3) & 7
                    s = (t & 7) * _L
                    ids16 = ids_v[cur, ti, jl, pl.ds(s, _L)]
                    for d in range(_D):
                        g = plsc.load_gather(tab_v, [ids16, dvecs[d]])
                        out_v[cur, jl, ti, d, pl.ds(s, _L)] = g

                out_dma(c, cur).start()

            return carry

        lax.fori_loop(0, _MAXU, unit_body, 0)
        for k in range(_MAXU - 2, _MAXU):
            @pl.when(valid(k))
            def _():
                out_dma(k, k & 1).wait()

    return emb


_emb = _make_emb()


@jax.jit
def kernel(input_ids, table):
    ids4 = input_ids.T.reshape(_JH, 8, _IH, 128).transpose(0, 2, 1, 3)
    out4 = _emb(table, ids4)
    return out4.transpose(1, 3, 0, 2).reshape(_ROWS, _COLS, _D)


# speedup vs baseline: 5.3251x; 5.3251x over previous
import functools

import jax
import jax.numpy as jnp
from jax import lax
from jax.experimental import pallas as pl
from jax.experimental.pallas import tpu as pltpu
from jax.experimental.pallas import tpu_sc as plsc

_INFO = plsc.get_sparse_core_info()
_NC = _INFO.num_cores
_NS = _INFO.num_subcores
_L = _INFO.num_lanes
_NW = _NC * _NS

_ROWS, _COLS = 16384, 200
_D = 4
_JH = _COLS // 8
_IH = _ROWS // 128
_NT = 8
_NUNITS = _JH * (_IH // _NT)
_MAXU = -(-_NUNITS // _NW)


def _make_emb():
    mesh = plsc.VectorSubcoreMesh(core_axis_name="c", subcore_axis_name="s")

    @functools.partial(
        pl.kernel,
        mesh=mesh,
        out_type=jax.ShapeDtypeStruct((_COLS, _IH, _D, 128), jnp.float32),
        compiler_params=pltpu.CompilerParams(needs_layout_passes=False),
        scratch_types=[
            pltpu.VMEM((8, _D), jnp.float32),
            pltpu.VMEM((8 * _D * _L,), jnp.float32),
            pltpu.VMEM((2, _NT, 8, 128), jnp.int32),
            pltpu.VMEM((2, 8, _NT, _D, 128), jnp.float32),
            pltpu.SemaphoreType.DMA((2,)),
            pltpu.SemaphoreType.DMA((2,)),
        ],
    )
    def emb(tab_hbm, ids_hbm, out_hbm, tab_v, tab_r, ids_v, out_v, isem, osem):
        wid = lax.axis_index("s") * _NC + lax.axis_index("c")
        pltpu.sync_copy(tab_hbm, tab_v)
        for k in range(8):
            for d in range(_D):
                val = plsc.load_gather(
                    tab_v, [jnp.full((_L,), k, jnp.int32),
                            jnp.full((_L,), d, jnp.int32)]
                )
                tab_r[pl.ds((k * _D + d) * _L, _L)] = val
        cvecs = [lax.iota(jnp.int32, _L) + d * _L for d in range(_D)]

        def valid(c):
            return c * _NW + wid < _NUNITS

        def ids_dma(c, buf):
            u = c * _NW + wid
            jh = u >> 4
            q = u & 15
            return pltpu.make_async_copy(
                ids_hbm.at[jh, pl.ds(q * _NT, _NT)], ids_v.at[buf], isem.at[buf]
            )

        def out_dma(c, buf):
            u = c * _NW + wid
            jh = u >> 4
            q = u & 15
            return pltpu.make_async_copy(
                out_v.at[buf],
                out_hbm.at[pl.ds(jh * 8, 8), pl.ds(q * _NT, _NT)],
                osem.at[buf],
            )

        ids_dma(0, 0).start()

        def unit_body(c, carry):
            cur = c & 1

            @pl.when(valid(c + 1))
            def _():
                ids_dma(c + 1, 1 - cur).start()

            @pl.when(valid(c))
            def _():
                ids_dma(c, cur).wait()

                @pl.when(c >= 2)
                def _():
                    out_dma(c - 2, cur).wait()

                @plsc.parallel_loop(0, _NT * 64, unroll=8)
                def body(t):
                    ti = t >> 6
                    jl = (t >> 3) & 7
                    s = (t & 7) * _L
                    ids16 = ids_v[cur, ti, jl, pl.ds(s, _L)]
                    base = ids16 * (_D * _L)
                    for d in range(_D):
                        g = plsc.load_gather(tab_r, [base + cvecs[d]])
                        out_v[cur, jl, ti, d, pl.ds(s, _L)] = g

                out_dma(c, cur).start()

            return carry

        lax.fori_loop(0, _MAXU, unit_body, 0)
        for k in range(_MAXU - 2, _MAXU):
            @pl.when(valid(k))
            def _():
                out_dma(k, k & 1).wait()

    return emb


_emb = _make_emb()


@jax.jit
def kernel(input_ids, table):
    ids4 = input_ids.T.reshape(_JH, 8, _IH, 128).transpose(0, 2, 1, 3)
    out4 = _emb(table, ids4)
    return out4.transpose(1, 3, 0, 2).reshape(_ROWS, _COLS, _D)
